# baseline (speedup 1.0000x reference)
import jax
import jax.numpy as jnp
from jax import lax
from jax.experimental import pallas as pl
from jax.experimental.pallas import tpu as pltpu
from jax.experimental.pallas import tpu_sc as plsc

B = 16384
HIST = 50
D = 32
GLOBAL_MEAN = 3.5

NC = 2
NS = 16
NW = NC * NS
RPW = B // NW
CH = 2
NCH = RPW // CH
PQ_CH = 128
NPQ = RPW // PQ_CH
TAB = 64


def _sc_body(sid_hbm, pid_hbm, ip_hbm, len_hbm, P_hbm, Q_hbm, Y_hbm, bs_hbm,
             bp_hbm, tab_hbm, part_hbm, bsum_hbm,
             ip_v, sid_v, pid_v, len_v, pbuf, qbuf, bsbuf, bpbuf, bsflat,
             combo, ybuf, out_v, sem_y, sem_misc):
    wid = lax.axis_index("s") * NC + lax.axis_index("c")
    base = wid * RPW

    pltpu.sync_copy(ip_hbm.at[pl.ds(wid * NCH, NCH)], ip_v)
    for k in range(NPQ):
        hsl = pl.ds(base + k * PQ_CH, PQ_CH)
        pltpu.sync_copy(sid_hbm.at[hsl], sid_v.at[k])
        pltpu.sync_copy(pid_hbm.at[hsl], pid_v.at[k])
        pltpu.sync_copy(len_hbm.at[hsl], len_v.at[k])

    for k in range(NPQ):
        sl = pl.ds(k * PQ_CH, PQ_CH)
        pltpu.async_copy(P_hbm.at[sid_v.at[k]], pbuf.at[sl], sem_misc)
        pltpu.async_copy(Q_hbm.at[pid_v.at[k]], qbuf.at[sl], sem_misc)
        pltpu.async_copy(bs_hbm.at[sid_v.at[pl.ds(k, 1)]], bsbuf.at[pl.ds(k, 1)], sem_misc)
        pltpu.async_copy(bp_hbm.at[pid_v.at[pl.ds(k, 1)]], bpbuf.at[pl.ds(k, 1)], sem_misc)
        pltpu.async_copy(tab_hbm.at[len_v.at[k]], combo.at[sl], sem_misc)

    pltpu.async_copy(Y_hbm.at[ip_v.at[0]], ybuf.at[0], sem_y)
    pltpu.async_copy(Y_hbm.at[ip_v.at[1]], ybuf.at[1], sem_y)

    for k in range(NPQ):
        sl = pl.ds(k * PQ_CH, PQ_CH)
        pltpu.make_async_copy(P_hbm.at[sid_v.at[k]], pbuf.at[sl], sem_misc).wait()
        pltpu.make_async_copy(Q_hbm.at[pid_v.at[k]], qbuf.at[sl], sem_misc).wait()
        pltpu.make_async_copy(bs_hbm.at[sid_v.at[pl.ds(k, 1)]], bsbuf.at[pl.ds(k, 1)], sem_misc).wait()
        pltpu.make_async_copy(bp_hbm.at[pid_v.at[pl.ds(k, 1)]], bpbuf.at[pl.ds(k, 1)], sem_misc).wait()
        pltpu.make_async_copy(tab_hbm.at[len_v.at[k]], combo.at[sl], sem_misc).wait()

    def pair_body(i, carry):
        for par in range(2):
            c = 2 * i + par
            yb = ybuf.at[par]
            pltpu.make_async_copy(Y_hbm.at[ip_v.at[0]], yb, sem_y).wait()
            for t in range(CH):
                r = c * CH + t
                lenf = combo[r, pl.ds(0, 16)]
                isq = combo[r, pl.ds(16, 16)]
                acc0 = yb[t * HIST, pl.ds(0, 16)]
                acc1 = yb[t * HIST, pl.ds(16, 16)]
                for j in range(1, HIST):
                    y0 = yb[t * HIST + j, pl.ds(0, 16)]
                    y1 = yb[t * HIST + j, pl.ds(16, 16)]
                    m = jnp.full((16,), float(j), jnp.float32) < lenf
                    acc0 = acc0 + jnp.where(m, y0, 0.0)
                    acc1 = acc1 + jnp.where(m, y1, 0.0)
                p0 = pbuf[r, pl.ds(0, 16)]
                p1 = pbuf[r, pl.ds(16, 16)]
                q0 = qbuf[r, pl.ds(0, 16)]
                q1 = qbuf[r, pl.ds(16, 16)]
                out_v[r, pl.ds(0, 16)] = q0 * (p0 + acc0 * isq)
                out_v[r, pl.ds(16, 16)] = q1 * (p1 + acc1 * isq)
            nxt = jnp.minimum(c + 2, NCH - 1)
            pltpu.async_copy(Y_hbm.at[ip_v.at[nxt]], yb, sem_y)
        return carry

    lax.fori_loop(0, NCH // 2, pair_body, 0)

    pltpu.make_async_copy(Y_hbm.at[ip_v.at[0]], ybuf.at[0], sem_y).wait()
    pltpu.make_async_copy(Y_hbm.at[ip_v.at[0]], ybuf.at[1], sem_y).wait()

    pltpu.sync_copy(out_v, part_hbm.at[pl.ds(base, RPW)])
    for k in range(NPQ):
        for g in range(PQ_CH // 16):
            sl = pl.ds(g * 16, 16)
            bsflat[pl.ds(k * PQ_CH + g * 16, 16)] = (
                bsbuf[k, sl] + bpbuf[k, sl])
    pltpu.sync_copy(bsflat, bsum_hbm.at[pl.ds(base, RPW)])


DT_PB = 4096


def _detile_body(xt_ref, o_ref):
    x = xt_ref[...].reshape(D, DT_PB // 4, 4)
    o_ref[...] = jnp.transpose(x, (1, 2, 0)).reshape(DT_PB // 4, 4 * D)


def _detile(table_t):
    n = table_t.shape[1]
    out = pl.pallas_call(
        _detile_body,
        grid=((n + DT_PB - 1) // DT_PB,),
        in_specs=[pl.BlockSpec((D, DT_PB), lambda i: (0, i))],
        out_specs=pl.BlockSpec((DT_PB // 4, 4 * D), lambda i: (i, 0)),
        out_shape=jax.ShapeDtypeStruct((n // 4, 4 * D), jnp.float32),
    )(table_t)
    return out.reshape(n, D)


def _tc_body(part_ref, bsum_ref, o_ref):
    o_ref[...] = (GLOBAL_MEAN + bsum_ref[...]
                  + jnp.sum(part_ref[...], axis=-1))


@jax.jit
def kernel(SIDs, PIDs, implicit_PIDs, implicit_lengths, P, Q, Y, Bs, Bp):
    len32 = implicit_lengths.astype(jnp.int32)
    v = jnp.arange(TAB, dtype=jnp.float32)
    isq = 1.0 / (jnp.sqrt(v) + 1e-9)
    tab = jnp.concatenate(
        [jnp.broadcast_to(v[:, None], (TAB, 16)),
         jnp.broadcast_to(isq[:, None], (TAB, 16))], axis=1)

    mesh = plsc.VectorSubcoreMesh(core_axis_name="c", subcore_axis_name="s",
                                  num_cores=NC, num_subcores=NS)
    sc = pl.kernel(
        _sc_body,
        out_type=(jax.ShapeDtypeStruct((B, D), jnp.float32),
                  jax.ShapeDtypeStruct((B,), jnp.float32)),
        mesh=mesh,
        compiler_params=pltpu.CompilerParams(use_tc_tiling_on_sc=False),
        scratch_types=[
            pltpu.VMEM((NCH, CH * HIST), jnp.int32),
            pltpu.VMEM((NPQ, PQ_CH), jnp.int32),
            pltpu.VMEM((NPQ, PQ_CH), jnp.int32),
            pltpu.VMEM((NPQ, PQ_CH), jnp.int32),
            pltpu.VMEM((RPW, D), jnp.float32),
            pltpu.VMEM((RPW, D), jnp.float32),
            pltpu.VMEM((NPQ, PQ_CH), jnp.float32),
            pltpu.VMEM((NPQ, PQ_CH), jnp.float32),
            pltpu.VMEM((RPW,), jnp.float32),
            pltpu.VMEM((RPW, D), jnp.float32),
            pltpu.VMEM((2, CH * HIST, D), jnp.float32),
            pltpu.VMEM((RPW, D), jnp.float32),
            pltpu.SemaphoreType.DMA,
            pltpu.SemaphoreType.DMA,
        ],
    )
    ip2 = implicit_PIDs.reshape(NW * NCH, CH * HIST)
    Qd = _detile(Q.T)
    Yd = _detile(Y.T)
    part, bsum = sc(SIDs, PIDs, ip2, len32, P, Qd, Yd, Bs.T, Bp.T, tab)

    out = pl.pallas_call(
        _tc_body,
        grid=(8,),
        in_specs=[pl.BlockSpec((16, 128, D), lambda i: (i, 0, 0)),
                  pl.BlockSpec((16, 128), lambda i: (i, 0))],
        out_specs=pl.BlockSpec((16, 128), lambda i: (i, 0)),
        out_shape=jax.ShapeDtypeStruct((128, 128), jnp.float32),
    )(part.reshape(128, 128, D), bsum.reshape(128, 128))
    return out.reshape(B)


# speedup vs baseline: 4.8183x; 4.8183x over previous
import jax
import jax.numpy as jnp
from jax import lax
from jax.experimental import pallas as pl
from jax.experimental.pallas import tpu as pltpu
from jax.experimental.pallas import tpu_sc as plsc

B = 16384
HIST = 50
D = 32
GLOBAL_MEAN = 3.5

NC = 2
NS = 16
NW = NC * NS
RPW = B // NW
CH = 2
NCH = RPW // CH
PQ_CH = 128
NPQ = RPW // PQ_CH
TAB = 64


def _sc_body(sid_hbm, pid_hbm, ip_hbm, len_hbm, P_hbm, Q_hbm, Y_hbm, bs_hbm,
             bp_hbm, tab_hbm, part_hbm, bsum_hbm,
             ip_v, sid_v, pid_v, len_v, pbuf, qbuf, bsbuf, bpbuf, bsflat,
             combo, ybuf, out_v, sem_y, sem_misc):
    wid = lax.axis_index("s") * NC + lax.axis_index("c")
    base = wid * RPW

    pltpu.sync_copy(ip_hbm.at[pl.ds(wid * NCH, NCH)], ip_v)
    for k in range(NPQ):
        hsl = pl.ds(base + k * PQ_CH, PQ_CH)
        pltpu.sync_copy(sid_hbm.at[hsl], sid_v.at[k])
        pltpu.sync_copy(pid_hbm.at[hsl], pid_v.at[k])
        pltpu.sync_copy(len_hbm.at[hsl], len_v.at[k])

    for k in range(NPQ):
        sl = pl.ds(k * PQ_CH, PQ_CH)
        pltpu.async_copy(P_hbm.at[sid_v.at[k]], pbuf.at[sl], sem_misc)
        pltpu.async_copy(Q_hbm.at[pid_v.at[k]], qbuf.at[sl], sem_misc)
        pltpu.async_copy(bs_hbm.at[sid_v.at[pl.ds(k, 1)]], bsbuf.at[pl.ds(k, 1)], sem_misc)
        pltpu.async_copy(bp_hbm.at[pid_v.at[pl.ds(k, 1)]], bpbuf.at[pl.ds(k, 1)], sem_misc)
        pltpu.async_copy(tab_hbm.at[len_v.at[k]], combo.at[sl], sem_misc)

    pltpu.async_copy(Y_hbm.at[ip_v.at[0]], ybuf.at[0], sem_y)
    pltpu.async_copy(Y_hbm.at[ip_v.at[1]], ybuf.at[1], sem_y)

    for k in range(NPQ):
        sl = pl.ds(k * PQ_CH, PQ_CH)
        pltpu.make_async_copy(P_hbm.at[sid_v.at[k]], pbuf.at[sl], sem_misc).wait()
        pltpu.make_async_copy(Q_hbm.at[pid_v.at[k]], qbuf.at[sl], sem_misc).wait()
        pltpu.make_async_copy(bs_hbm.at[sid_v.at[pl.ds(k, 1)]], bsbuf.at[pl.ds(k, 1)], sem_misc).wait()
        pltpu.make_async_copy(bp_hbm.at[pid_v.at[pl.ds(k, 1)]], bpbuf.at[pl.ds(k, 1)], sem_misc).wait()
        pltpu.make_async_copy(tab_hbm.at[len_v.at[k]], combo.at[sl], sem_misc).wait()

    def pair_body(i, carry):
        for par in range(2):
            c = 2 * i + par
            yb = ybuf.at[par]
            pltpu.make_async_copy(Y_hbm.at[ip_v.at[0]], yb, sem_y).wait()
            for t in range(CH):
                r = c * CH + t
                lenf = combo[r, pl.ds(0, 16)]
                isq = combo[r, pl.ds(16, 16)]
                acc0 = yb[t * HIST, pl.ds(0, 16)]
                acc1 = yb[t * HIST, pl.ds(16, 16)]
                for j in range(1, HIST):
                    y0 = yb[t * HIST + j, pl.ds(0, 16)]
                    y1 = yb[t * HIST + j, pl.ds(16, 16)]
                    m = jnp.full((16,), float(j), jnp.float32) < lenf
                    acc0 = acc0 + jnp.where(m, y0, 0.0)
                    acc1 = acc1 + jnp.where(m, y1, 0.0)
                p0 = pbuf[r, pl.ds(0, 16)]
                p1 = pbuf[r, pl.ds(16, 16)]
                q0 = qbuf[r, pl.ds(0, 16)]
                q1 = qbuf[r, pl.ds(16, 16)]
                out_v[r, pl.ds(0, 16)] = q0 * (p0 + acc0 * isq)
                out_v[r, pl.ds(16, 16)] = q1 * (p1 + acc1 * isq)
            nxt = jnp.minimum(c + 2, NCH - 1)
            pltpu.async_copy(Y_hbm.at[ip_v.at[nxt]], yb, sem_y)
        return carry

    lax.fori_loop(0, NCH // 2, pair_body, 0)

    pltpu.make_async_copy(Y_hbm.at[ip_v.at[0]], ybuf.at[0], sem_y).wait()
    pltpu.make_async_copy(Y_hbm.at[ip_v.at[0]], ybuf.at[1], sem_y).wait()

    pltpu.sync_copy(out_v, part_hbm.at[pl.ds(base, RPW)])
    for k in range(NPQ):
        for g in range(PQ_CH // 16):
            sl = pl.ds(g * 16, 16)
            bsflat[pl.ds(k * PQ_CH + g * 16, 16)] = (
                bsbuf[k, sl] + bpbuf[k, sl])
    pltpu.sync_copy(bsflat, bsum_hbm.at[pl.ds(base, RPW)])


def _tc_body(part_ref, bsum_ref, o_ref):
    o_ref[...] = (GLOBAL_MEAN + bsum_ref[...]
                  + jnp.sum(part_ref[...], axis=-1))


@jax.jit
def kernel(SIDs, PIDs, implicit_PIDs, implicit_lengths, P, Q, Y, Bs, Bp):
    len32 = implicit_lengths.astype(jnp.int32)
    v = jnp.arange(TAB, dtype=jnp.float32)
    isq = 1.0 / (jnp.sqrt(v) + 1e-9)
    tab = jnp.concatenate(
        [jnp.broadcast_to(v[:, None], (TAB, 16)),
         jnp.broadcast_to(isq[:, None], (TAB, 16))], axis=1)

    mesh = plsc.VectorSubcoreMesh(core_axis_name="c", subcore_axis_name="s",
                                  num_cores=NC, num_subcores=NS)
    sc = pl.kernel(
        _sc_body,
        out_type=(jax.ShapeDtypeStruct((B, D), jnp.float32),
                  jax.ShapeDtypeStruct((B,), jnp.float32)),
        mesh=mesh,
        compiler_params=pltpu.CompilerParams(use_tc_tiling_on_sc=False),
        scratch_types=[
            pltpu.VMEM((NCH, CH * HIST), jnp.int32),
            pltpu.VMEM((NPQ, PQ_CH), jnp.int32),
            pltpu.VMEM((NPQ, PQ_CH), jnp.int32),
            pltpu.VMEM((NPQ, PQ_CH), jnp.int32),
            pltpu.VMEM((RPW, D), jnp.float32),
            pltpu.VMEM((RPW, D), jnp.float32),
            pltpu.VMEM((NPQ, PQ_CH), jnp.float32),
            pltpu.VMEM((NPQ, PQ_CH), jnp.float32),
            pltpu.VMEM((RPW,), jnp.float32),
            pltpu.VMEM((RPW, D), jnp.float32),
            pltpu.VMEM((2, CH * HIST, D), jnp.float32),
            pltpu.VMEM((RPW, D), jnp.float32),
            pltpu.SemaphoreType.DMA,
            pltpu.SemaphoreType.DMA,
        ],
    )
    ip2 = implicit_PIDs.reshape(NW * NCH, CH * HIST)
    part, bsum = sc(SIDs, PIDs, ip2, len32, P, Q, Y, Bs.T, Bp.T, tab)

    out = pl.pallas_call(
        _tc_body,
        grid=(8,),
        in_specs=[pl.BlockSpec((16, 128, D), lambda i: (i, 0, 0)),
                  pl.BlockSpec((16, 128), lambda i: (i, 0))],
        out_specs=pl.BlockSpec((16, 128), lambda i: (i, 0)),
        out_shape=jax.ShapeDtypeStruct((128, 128), jnp.float32),
    )(part.reshape(128, 128, D), bsum.reshape(128, 128))
    return out.reshape(B)


# speedup vs baseline: 5.4189x; 1.1246x over previous
import jax
import jax.numpy as jnp
from jax import lax
from jax.experimental import pallas as pl
from jax.experimental.pallas import tpu as pltpu
from jax.experimental.pallas import tpu_sc as plsc

B = 16384
HIST = 50
D = 32
GLOBAL_MEAN = 3.5

NC = 2
NS = 16
NW = NC * NS
RPW = B // NW
CH = 2
NCH = RPW // CH
PQ_CH = 128
NPQ = RPW // PQ_CH
TAB = 64


def _sc_body_a(sid_hbm, pid_hbm, ip_hbm, len_hbm, P_hbm, Y_hbm, bs_hbm,
               bp_hbm, tab_hbm, svec_hbm, bsum_hbm,
               ip_v, sid_v, pid_v, len_v, pbuf, bsbuf, bpbuf, bsflat, combo,
               ybuf, out_v, sem_y, sem_misc):
    wid = lax.axis_index("s") * NC + lax.axis_index("c")
    base = wid * RPW

    pltpu.sync_copy(ip_hbm.at[pl.ds(wid * NCH, NCH)], ip_v)
    for k in range(NPQ):
        hsl = pl.ds(base + k * PQ_CH, PQ_CH)
        pltpu.sync_copy(sid_hbm.at[hsl], sid_v.at[k])
        pltpu.sync_copy(pid_hbm.at[hsl], pid_v.at[k])
        pltpu.sync_copy(len_hbm.at[hsl], len_v.at[k])

    for k in range(NPQ):
        sl = pl.ds(k * PQ_CH, PQ_CH)
        k1 = pl.ds(k, 1)
        pltpu.async_copy(P_hbm.at[sid_v.at[k]], pbuf.at[sl], sem_misc)
        pltpu.async_copy(bs_hbm.at[sid_v.at[k1]], bsbuf.at[k1], sem_misc)
        pltpu.async_copy(bp_hbm.at[pid_v.at[k1]], bpbuf.at[k1], sem_misc)
        pltpu.async_copy(tab_hbm.at[len_v.at[k]], combo.at[sl], sem_misc)

    pltpu.async_copy(Y_hbm.at[ip_v.at[0]], ybuf.at[0], sem_y)
    pltpu.async_copy(Y_hbm.at[ip_v.at[1]], ybuf.at[1], sem_y)

    for k in range(NPQ):
        sl = pl.ds(k * PQ_CH, PQ_CH)
        k1 = pl.ds(k, 1)
        pltpu.make_async_copy(P_hbm.at[sid_v.at[k]], pbuf.at[sl], sem_misc).wait()
        pltpu.make_async_copy(bs_hbm.at[sid_v.at[k1]], bsbuf.at[k1], sem_misc).wait()
        pltpu.make_async_copy(bp_hbm.at[pid_v.at[k1]], bpbuf.at[k1], sem_misc).wait()
        pltpu.make_async_copy(tab_hbm.at[len_v.at[k]], combo.at[sl], sem_misc).wait()

    def pair_body(i, carry):
        for par in range(2):
            c = 2 * i + par
            yb = ybuf.at[par]
            pltpu.make_async_copy(Y_hbm.at[ip_v.at[0]], yb, sem_y).wait()
            for t in range(CH):
                r = c * CH + t
                lenf = combo[r, pl.ds(0, 16)]
                isq = combo[r, pl.ds(16, 16)]
                acc0 = yb[t * HIST, pl.ds(0, 16)]
                acc1 = yb[t * HIST, pl.ds(16, 16)]
                for j in range(1, HIST):
                    y0 = yb[t * HIST + j, pl.ds(0, 16)]
                    y1 = yb[t * HIST + j, pl.ds(16, 16)]
                    m = jnp.full((16,), float(j), jnp.float32) < lenf
                    acc0 = acc0 + jnp.where(m, y0, 0.0)
                    acc1 = acc1 + jnp.where(m, y1, 0.0)
                p0 = pbuf[r, pl.ds(0, 16)]
                p1 = pbuf[r, pl.ds(16, 16)]
                out_v[r, pl.ds(0, 16)] = p0 + acc0 * isq
                out_v[r, pl.ds(16, 16)] = p1 + acc1 * isq
            nxt = jnp.minimum(c + 2, NCH - 1)
            pltpu.async_copy(Y_hbm.at[ip_v.at[nxt]], yb, sem_y)
        return carry

    lax.fori_loop(0, NCH // 2, pair_body, 0)

    pltpu.make_async_copy(Y_hbm.at[ip_v.at[0]], ybuf.at[0], sem_y).wait()
    pltpu.make_async_copy(Y_hbm.at[ip_v.at[0]], ybuf.at[1], sem_y).wait()

    pltpu.sync_copy(out_v, svec_hbm.at[pl.ds(base, RPW)])
    for k in range(NPQ):
        for g in range(PQ_CH // 16):
            sl = pl.ds(g * 16, 16)
            bsflat[pl.ds(k * PQ_CH + g * 16, 16)] = (
                bsbuf[k, sl] + bpbuf[k, sl])
    pltpu.sync_copy(bsflat, bsum_hbm.at[pl.ds(base, RPW)])


def _sc_body_b(pid_hbm, Q_hbm, svec_hbm, part_hbm,
               pid_v, qbuf, svbuf, sem_misc):
    wid = lax.axis_index("s") * NC + lax.axis_index("c")
    base = wid * RPW

    for k in range(NPQ):
        pltpu.sync_copy(pid_hbm.at[pl.ds(base + k * PQ_CH, PQ_CH)],
                        pid_v.at[k])
    for k in range(NPQ):
        sl = pl.ds(k * PQ_CH, PQ_CH)
        pltpu.async_copy(Q_hbm.at[pid_v.at[k]], qbuf.at[sl], sem_misc)
    pltpu.sync_copy(svec_hbm.at[pl.ds(base, RPW)], svbuf)
    for k in range(NPQ):
        sl = pl.ds(k * PQ_CH, PQ_CH)
        pltpu.make_async_copy(Q_hbm.at[pid_v.at[k]], qbuf.at[sl], sem_misc).wait()

    def mul_body(i, carry):
        for t in range(2):
            r = 2 * i + t
            qbuf[r, pl.ds(0, 16)] = (qbuf[r, pl.ds(0, 16)]
                                     * svbuf[r, pl.ds(0, 16)])
            qbuf[r, pl.ds(16, 16)] = (qbuf[r, pl.ds(16, 16)]
                                      * svbuf[r, pl.ds(16, 16)])
        return carry

    lax.fori_loop(0, RPW // 2, mul_body, 0)
    pltpu.sync_copy(qbuf, part_hbm.at[pl.ds(base, RPW)])


def _tc_body(part_ref, bsum_ref, o_ref):
    o_ref[...] = (GLOBAL_MEAN + bsum_ref[...]
                  + jnp.sum(part_ref[...], axis=-1))


@jax.jit
def kernel(SIDs, PIDs, implicit_PIDs, implicit_lengths, P, Q, Y, Bs, Bp):
    len32 = implicit_lengths.astype(jnp.int32)
    v = jnp.arange(TAB, dtype=jnp.float32)
    isq = 1.0 / (jnp.sqrt(v) + 1e-9)
    tab = jnp.concatenate(
        [jnp.broadcast_to(v[:, None], (TAB, 16)),
         jnp.broadcast_to(isq[:, None], (TAB, 16))], axis=1)

    mesh = plsc.VectorSubcoreMesh(core_axis_name="c", subcore_axis_name="s",
                                  num_cores=NC, num_subcores=NS)
    sc_a = pl.kernel(
        _sc_body_a,
        out_type=(jax.ShapeDtypeStruct((B, D), jnp.float32),
                  jax.ShapeDtypeStruct((B,), jnp.float32)),
        mesh=mesh,
        compiler_params=pltpu.CompilerParams(use_tc_tiling_on_sc=False),
        scratch_types=[
            pltpu.VMEM((NCH, CH * HIST), jnp.int32),
            pltpu.VMEM((NPQ, PQ_CH), jnp.int32),
            pltpu.VMEM((NPQ, PQ_CH), jnp.int32),
            pltpu.VMEM((NPQ, PQ_CH), jnp.int32),
            pltpu.VMEM((RPW, D), jnp.float32),
            pltpu.VMEM((NPQ, PQ_CH), jnp.float32),
            pltpu.VMEM((NPQ, PQ_CH), jnp.float32),
            pltpu.VMEM((RPW,), jnp.float32),
            pltpu.VMEM((RPW, D), jnp.float32),
            pltpu.VMEM((2, CH * HIST, D), jnp.float32),
            pltpu.VMEM((RPW, D), jnp.float32),
            pltpu.SemaphoreType.DMA,
            pltpu.SemaphoreType.DMA,
        ],
    )
    sc_b = pl.kernel(
        _sc_body_b,
        out_type=jax.ShapeDtypeStruct((B, D), jnp.float32),
        mesh=mesh,
        compiler_params=pltpu.CompilerParams(use_tc_tiling_on_sc=False),
        scratch_types=[
            pltpu.VMEM((NPQ, PQ_CH), jnp.int32),
            pltpu.VMEM((RPW, D), jnp.float32),
            pltpu.VMEM((RPW, D), jnp.float32),
            pltpu.SemaphoreType.DMA,
        ],
    )
    ip2 = implicit_PIDs.reshape(NW * NCH, CH * HIST)
    svec, bsum = sc_a(SIDs, PIDs, ip2, len32, P, Y, Bs.T, Bp.T, tab)
    part = sc_b(PIDs, Q, svec)

    out = pl.pallas_call(
        _tc_body,
        grid=(8,),
        in_specs=[pl.BlockSpec((16, 128, D), lambda i: (i, 0, 0)),
                  pl.BlockSpec((16, 128), lambda i: (i, 0))],
        out_specs=pl.BlockSpec((16, 128), lambda i: (i, 0)),
        out_shape=jax.ShapeDtypeStruct((128, 128), jnp.float32),
    )(part.reshape(128, 128, D), bsum.reshape(128, 128))
    return out.reshape(B)
